# baseline (speedup 1.0000x reference)
import functools

import jax
import jax.numpy as jnp
from jax import lax
from jax.experimental import pallas as pl
from jax.experimental.pallas import tpu as pltpu
from jax.experimental.pallas import tpu_sc as plsc

NW = 32
E_PER_W = 10000
CHUNKS = 125
C = 80
D = 128
GROUPS = C // 16


def _body(chem, dis, src, dst, out, src_v, dst_v, chem_v, dis_v, out_v, sem):
    wid = lax.axis_index("s") * 2 + lax.axis_index("c")
    pltpu.sync_copy(src.at[wid], src_v)
    pltpu.sync_copy(dst.at[wid], dst_v)

    lanes = lax.iota(jnp.int32, 16)

    def chunk_body(c, _):
        cp1 = pltpu.async_copy(chem.at[src_v.at[c]], chem_v, sem)
        cp2 = pltpu.async_copy(dis.at[dst_v.at[c]], dis_v, sem)
        cp1.wait()
        cp2.wait()

        def group(g, _):
            rows = g * 16 + lanes

            def fstep(f8, acc):
                base = jnp.full((16,), 0, jnp.int32) + f8 * 8
                for j in range(8):
                    col = base + j
                    a = plsc.load_gather(chem_v, [rows, col])
                    b = plsc.load_gather(dis_v, [rows, col])
                    acc = acc + a * b
                return acc

            acc = lax.fori_loop(0, D // 8, fstep, jnp.zeros((16,), jnp.float32))
            out_v[pl.ds(c * C + g * 16, 16)] = acc
            return 0

        lax.fori_loop(0, GROUPS, group, 0)
        return 0

    lax.fori_loop(0, CHUNKS, chunk_body, 0)
    pltpu.sync_copy(out_v, out.at[wid])


@jax.jit
def _run(chemical, disease, src, dst):
    kfn = functools.partial(
        pl.kernel,
        mesh=plsc.VectorSubcoreMesh(core_axis_name="c", subcore_axis_name="s"),
        compiler_params=pltpu.CompilerParams(needs_layout_passes=False),
        out_type=jax.ShapeDtypeStruct((NW, E_PER_W), jnp.float32),
        scratch_types=[
            pltpu.VMEM((CHUNKS, C), jnp.int32),
            pltpu.VMEM((CHUNKS, C), jnp.int32),
            pltpu.VMEM((C, D), jnp.float32),
            pltpu.VMEM((C, D), jnp.float32),
            pltpu.VMEM((E_PER_W,), jnp.float32),
            pltpu.SemaphoreType.DMA,
        ],
    )(_body)
    return kfn(chemical, disease, src, dst)


def kernel(chemical, disease, edge_label_index):
    idx = edge_label_index.astype(jnp.int32)
    src = idx[0].reshape(NW, CHUNKS, C)
    dst = idx[1].reshape(NW, CHUNKS, C)
    out = _run(chemical, disease, src, dst)
    return out.reshape(NW * E_PER_W)


# speedup vs baseline: 1.1452x; 1.1452x over previous
import functools

import jax
import jax.numpy as jnp
from jax import lax
from jax.experimental import pallas as pl
from jax.experimental.pallas import tpu as pltpu
from jax.experimental.pallas import tpu_sc as plsc

NW = 32
E_PER_W = 10000
CHUNKS = 125
C = 80
D = 128
GROUPS = C // 16
NBUF = 4


def _body(chem, dis, src, dst, out, src_v, dst_v,
          cb0, cb1, cb2, cb3, db0, db1, db2, db3, out_v,
          sem0, sem1, sem2, sem3):
    cbufs = (cb0, cb1, cb2, cb3)
    dbufs = (db0, db1, db2, db3)
    sems = (sem0, sem1, sem2, sem3)

    wid = lax.axis_index("s") * 2 + lax.axis_index("c")
    pltpu.sync_copy(src.at[wid], src_v)
    pltpu.sync_copy(dst.at[wid], dst_v)

    lanes = lax.iota(jnp.int32, 16)

    def issue(c, b):
        pltpu.async_copy(chem.at[src_v.at[pl.ds(c * C, C)]], cbufs[b], sems[b])
        pltpu.async_copy(dis.at[dst_v.at[pl.ds(c * C, C)]], dbufs[b], sems[b])

    def drain(b):
        pltpu.make_async_copy(
            chem.at[src_v.at[pl.ds(0, C)]], cbufs[b], sems[b]).wait()
        pltpu.make_async_copy(
            dis.at[dst_v.at[pl.ds(0, C)]], dbufs[b], sems[b]).wait()

    def compute(c, b):
        def group(g, _):
            rows = g * 16 + lanes

            def fstep(f8, acc):
                base = jnp.full((16,), 0, jnp.int32) + f8 * 8
                for j in range(8):
                    col = base + j
                    a = plsc.load_gather(cbufs[b], [rows, col])
                    bb = plsc.load_gather(dbufs[b], [rows, col])
                    acc = acc + a * bb
                return acc

            acc = lax.fori_loop(0, D // 8, fstep, jnp.zeros((16,), jnp.float32))
            out_v[pl.ds(c * C + g * 16, 16)] = acc
            return 0

        lax.fori_loop(0, GROUPS, group, 0)

    for b in range(NBUF):
        issue(b, b)

    def step(k, _):
        for b in range(NBUF):
            c = k * NBUF + b
            drain(b)
            compute(c, b)

            @pl.when(c + NBUF < CHUNKS)
            def _():
                issue(c + NBUF, b)

        return 0

    lax.fori_loop(0, (CHUNKS - 1) // NBUF, step, 0)
    drain(0)
    compute(CHUNKS - 1, 0)
    pltpu.sync_copy(out_v, out.at[wid])


@jax.jit
def _run(chemical, disease, src, dst):
    kfn = functools.partial(
        pl.kernel,
        mesh=plsc.VectorSubcoreMesh(core_axis_name="c", subcore_axis_name="s"),
        compiler_params=pltpu.CompilerParams(needs_layout_passes=False),
        out_type=jax.ShapeDtypeStruct((NW, E_PER_W), jnp.float32),
        scratch_types=[
            pltpu.VMEM((E_PER_W,), jnp.int32),
            pltpu.VMEM((E_PER_W,), jnp.int32),
        ] + [pltpu.VMEM((C, D), jnp.float32)] * (2 * NBUF) + [
            pltpu.VMEM((E_PER_W,), jnp.float32),
        ] + [pltpu.SemaphoreType.DMA] * NBUF,
    )(_body)
    return kfn(chemical, disease, src, dst)


def kernel(chemical, disease, edge_label_index):
    idx = edge_label_index.astype(jnp.int32)
    src = idx[0].reshape(NW, E_PER_W)
    dst = idx[1].reshape(NW, E_PER_W)
    out = _run(chemical, disease, src, dst)
    return out.reshape(NW * E_PER_W)
